# baseline (speedup 1.0000x reference)
import functools

import jax
import jax.numpy as jnp
from jax import lax
from jax.experimental import pallas as pl
from jax.experimental.pallas import tpu as pltpu
from jax.experimental.pallas import tpu_sc as plsc

_R2 = 0.15 * 0.15
_MAXNBR = 32
_N = 10000
_NBATCH = 8
_NC = 2
_NS = 16
_NW = _NC * _NS
_WROWS = 320
_NPAD = _NW * _WROWS
_BUF = _N + 64

_INF = float("inf")
_IBIG = 2**31 - 1


def _lex_less(ka, va, kb, vb):
  return (ka < kb) | ((ka == kb) & (va < vb))


def _bitonic_split(ak, av, bk, bv):
  rbk = lax.rev(bk, (0,))
  rbv = lax.rev(bv, (0,))
  m = _lex_less(ak, av, rbk, rbv)
  lo_k = jnp.where(m, ak, rbk)
  lo_v = jnp.where(m, av, rbv)
  hi_k = jnp.where(m, rbk, ak)
  hi_v = jnp.where(m, rbv, av)
  return lo_k, lo_v, hi_k, hi_v


@functools.cache
def _build_radius_sc():
  mesh = plsc.VectorSubcoreMesh(core_axis_name="c", subcore_axis_name="s")

  @functools.partial(
      pl.kernel,
      out_type=(
          jax.ShapeDtypeStruct((_NPAD * _MAXNBR,), jnp.int32),
          jax.ShapeDtypeStruct((_NPAD * _MAXNBR,), jnp.int32),
          jax.ShapeDtypeStruct((_NPAD,), jnp.int32),
      ),
      mesh=mesh,
      scratch_types=[
          pltpu.VMEM((_N + 64,), jnp.float32),
          pltpu.VMEM((_N + 64,), jnp.float32),
          pltpu.VMEM((_N + 64,), jnp.float32),
          pltpu.VMEM((_NPAD,), jnp.int32),
          pltpu.VMEM((_WROWS + 16,), jnp.int32),
          pltpu.VMEM((_WROWS + 16,), jnp.int32),
          pltpu.VMEM((_BUF,), jnp.float32),
          pltpu.VMEM((_BUF,), jnp.int32),
          pltpu.VMEM((_WROWS * _MAXNBR,), jnp.int32),
          pltpu.VMEM((_WROWS * _MAXNBR,), jnp.int32),
          pltpu.VMEM((_WROWS,), jnp.int32),
      ],
      compiler_params=pltpu.CompilerParams(needs_layout_passes=False),
  )
  def _radius_sc(cx_h, cy_h, cz_h, b_h, src_h, dst_h, deg_h,
                 cxv, cyv, czv, bv, s_arr, e_arr, bufd, bufi,
                 srcb, dstb, degb):
    wid = lax.axis_index("s") * _NC + lax.axis_index("c")
    r0 = wid * _WROWS
    cnt = jnp.minimum(_WROWS, _N - r0)

    pltpu.sync_copy(cx_h, cxv.at[pl.ds(0, _N)])
    pltpu.sync_copy(cy_h, cyv.at[pl.ds(0, _N)])
    pltpu.sync_copy(cz_h, czv.at[pl.ds(0, _N)])
    pltpu.sync_copy(b_h, bv.at[pl.ds(0, _N)])

    iota = lax.iota(jnp.int32, 16)

    zf = jnp.zeros((16,), jnp.float32)
    bigb = jnp.full((16,), _NBATCH, jnp.int32)
    for q in range(_N, _N + 64, 16):
      cxv[pl.ds(q, 16)] = zf
      cyv[pl.ds(q, 16)] = zf
      czv[pl.ds(q, 16)] = zf
    for q in range(_N, _NPAD, 16):
      bv[pl.ds(q, 16)] = bigb

    def lower_bound(val):
      def bs(_, carry):
        lo, hi = carry
        mid = (lo + hi) // 2
        v = bv[pl.ds(mid, 16)][0]
        go = v < val
        lo2 = jnp.where(go, mid + 1, lo)
        hi2 = jnp.where(go, hi, mid)
        return lo2, hi2
      lo, _ = lax.fori_loop(0, 14, bs, (jnp.int32(0), jnp.int32(_N)))
      return lo

    lb = [jnp.int32(0)]
    for b in range(_NBATCH):
      lb.append(lower_bound(jnp.int32(b + 1)))

    for g in range(_WROWS // 16):
      bvec = bv[pl.ds(r0 + g * 16, 16)]
      svec = jnp.zeros((16,), jnp.int32)
      evec = jnp.zeros((16,), jnp.int32)
      for b in range(_NBATCH):
        svec = jnp.where(bvec == b, lb[b], svec)
        evec = jnp.where(bvec == b, lb[b + 1], evec)
      s_arr[pl.ds(g * 16, 16)] = svec
      e_arr[pl.ds(g * 16, 16)] = evec

    def row_body(r, carry):
      @pl.when(r < cnt)
      def _():
        i = r0 + r
        s = s_arr[pl.ds(r, 16)][0]
        e = e_arr[pl.ds(r, 16)][0]
        cxi = cxv[pl.ds(i, 16)][0]
        cyi = cyv[pl.ds(i, 16)][0]
        czi = czv[pl.ds(i, 16)][0]
        t0 = s // 16
        t1 = (e + 15) // 16
        seg_len = e - s

        def scan_blk(t, cvec):
          j0 = t * 16
          jv = j0 + iota
          x = cxv[pl.ds(j0, 16)]
          y = cyv[pl.ds(j0, 16)]
          z = czv[pl.ds(j0, 16)]
          dx = x - cxi
          dy = y - cyi
          dz = z - czi
          d2 = dx * dx + dy * dy + dz * dz
          inseg = (jv - s).astype(jnp.uint32) < seg_len.astype(jnp.uint32)
          m = inseg & (jv != i) & (d2 < _R2)
          m32 = m.astype(jnp.int32)
          cums = plsc.cumsum(m32)
          pos = cvec + cums - m32
          plsc.store_scatter(bufd, [pos], d2, mask=m)
          plsc.store_scatter(bufi, [pos], jv, mask=m)
          return cvec + plsc.all_reduce_population_count(m)

        C = plsc.parallel_loop(
            t0, t1, 1, unroll=8,
            carry=jnp.zeros((16,), jnp.int32))(scan_blk)[0]
        kc = jnp.minimum(C, _MAXNBR)

        inf_v = jnp.full((16,), _INF, jnp.float32)
        big_v = jnp.full((16,), _IBIG, jnp.int32)
        for p in range(3):
          bufd[pl.ds(C + p * 16, 16)] = inf_v
          bufi[pl.ds(C + p * 16, 16)] = big_v

        k0, v0 = plsc.sort_key_val(bufd[pl.ds(0, 16)], bufi[pl.ds(0, 16)])
        k1, v1 = plsc.sort_key_val(bufd[pl.ds(16, 16)], bufi[pl.ds(16, 16)])
        lo_k, lo_v, hi_k, hi_v = _bitonic_split(k0, v0, k1, v1)
        k0, v0 = plsc.sort_key_val(lo_k, lo_v)
        k1, v1 = plsc.sort_key_val(hi_k, hi_v)

        nblk = (C + 15) // 16

        def sel_blk(t, sel):
          s0, w0, s1, w1 = sel
          sk, sv = plsc.sort_key_val(bufd[pl.ds(t * 16, 16)],
                                     bufi[pl.ds(t * 16, 16)])
          a_k, a_v, rest_k, rest_v = _bitonic_split(s0, w0, sk, sv)
          s0n, w0n = plsc.sort_key_val(a_k, a_v)
          rk, rv = plsc.sort_key_val(rest_k, rest_v)
          c_k, c_v, _, _ = _bitonic_split(s1, w1, rk, rv)
          s1n, w1n = plsc.sort_key_val(c_k, c_v)
          return s0n, w0n, s1n, w1n

        k0, v0, k1, v1 = lax.fori_loop(2, nblk, sel_blk, (k0, v0, k1, v1))

        srcb[pl.ds(r * 32, 16)] = jnp.where(iota < kc, v0, -1)
        srcb[pl.ds(r * 32 + 16, 16)] = jnp.where(iota + 16 < kc, v1, -1)
        dstb[pl.ds(r * 32, 16)] = jnp.where(iota < kc, i, -1)
        dstb[pl.ds(r * 32 + 16, 16)] = jnp.where(iota + 16 < kc, i, -1)
        plsc.store_scatter(degb, [jnp.broadcast_to(r, (16,))],
                           jnp.broadcast_to(kc, (16,)), mask=iota == 0)
      return carry

    lax.fori_loop(0, _WROWS, row_body, 0)

    pltpu.sync_copy(srcb, src_h.at[pl.ds(r0 * 32, _WROWS * 32)])
    pltpu.sync_copy(dstb, dst_h.at[pl.ds(r0 * 32, _WROWS * 32)])
    pltpu.sync_copy(degb, deg_h.at[pl.ds(r0, _WROWS)])

  return _radius_sc


def kernel(node_coord_src, node_feature_src, batch_src):
  cx = node_coord_src[:, 0]
  cy = node_coord_src[:, 1]
  cz = node_coord_src[:, 2]
  src_p, dst_p, deg_p = _build_radius_sc()(cx, cy, cz, batch_src)
  edge_src = src_p[: _N * _MAXNBR]
  edge_dst = dst_p[: _N * _MAXNBR]
  degree = deg_p[:_N]
  return (node_feature_src, node_coord_src, edge_src, edge_dst, degree,
          batch_src)


# speedup vs baseline: 1.3383x; 1.3383x over previous
import functools

import jax
import jax.numpy as jnp
from jax import lax
from jax.experimental import pallas as pl
from jax.experimental.pallas import tpu as pltpu
from jax.experimental.pallas import tpu_sc as plsc

_R2 = 0.15 * 0.15
_MAXNBR = 32
_N = 10000
_NBATCH = 8
_NC = 2
_NS = 16
_NW = _NC * _NS
_WROWS = 320
_NPAD = _NW * _WROWS
_BUF = _N + 64

_INF = float("inf")
_IBIG = 2**31 - 1


def _lex_less(ka, va, kb, vb):
  return (ka < kb) | ((ka == kb) & (va < vb))


def _bitonic_split(ak, av, bk, bv):
  rbk = lax.rev(bk, (0,))
  rbv = lax.rev(bv, (0,))
  m = _lex_less(ak, av, rbk, rbv)
  lo_k = jnp.where(m, ak, rbk)
  lo_v = jnp.where(m, av, rbv)
  hi_k = jnp.where(m, rbk, ak)
  hi_v = jnp.where(m, rbv, av)
  return lo_k, lo_v, hi_k, hi_v


@functools.cache
def _build_radius_sc():
  mesh = plsc.VectorSubcoreMesh(core_axis_name="c", subcore_axis_name="s")

  @functools.partial(
      pl.kernel,
      out_type=(
          jax.ShapeDtypeStruct((_NPAD * _MAXNBR,), jnp.int32),
          jax.ShapeDtypeStruct((_NPAD * _MAXNBR,), jnp.int32),
          jax.ShapeDtypeStruct((_NPAD,), jnp.int32),
      ),
      mesh=mesh,
      scratch_types=[
          pltpu.VMEM((_N + 64,), jnp.float32),
          pltpu.VMEM((_N + 64,), jnp.float32),
          pltpu.VMEM((_N + 64,), jnp.float32),
          pltpu.VMEM((_NPAD,), jnp.int32),
          pltpu.VMEM((_WROWS + 16,), jnp.int32),
          pltpu.VMEM((_WROWS + 16,), jnp.int32),
          pltpu.VMEM((_BUF,), jnp.float32),
          pltpu.VMEM((_BUF,), jnp.int32),
          pltpu.VMEM((_WROWS * _MAXNBR,), jnp.int32),
          pltpu.VMEM((_WROWS * _MAXNBR,), jnp.int32),
          pltpu.VMEM((_WROWS,), jnp.int32),
      ],
      compiler_params=pltpu.CompilerParams(needs_layout_passes=False),
  )
  def _radius_sc(cx_h, cy_h, cz_h, b_h, src_h, dst_h, deg_h,
                 cxv, cyv, czv, bv, s_arr, e_arr, bufd, bufi,
                 srcb, dstb, degb):
    wid = lax.axis_index("s") * _NC + lax.axis_index("c")
    r0 = wid * _WROWS
    cnt = jnp.minimum(_WROWS, _N - r0)

    pltpu.sync_copy(cx_h, cxv.at[pl.ds(0, _N)])
    pltpu.sync_copy(cy_h, cyv.at[pl.ds(0, _N)])
    pltpu.sync_copy(cz_h, czv.at[pl.ds(0, _N)])
    pltpu.sync_copy(b_h, bv.at[pl.ds(0, _N)])

    iota = lax.iota(jnp.int32, 16)

    zf = jnp.zeros((16,), jnp.float32)
    bigb = jnp.full((16,), _NBATCH, jnp.int32)
    for q in range(_N, _N + 64, 16):
      cxv[pl.ds(q, 16)] = zf
      cyv[pl.ds(q, 16)] = zf
      czv[pl.ds(q, 16)] = zf
    for q in range(_N, _NPAD, 16):
      bv[pl.ds(q, 16)] = bigb

    def lower_bound(val):
      def bs(_, carry):
        lo, hi = carry
        mid = (lo + hi) // 2
        v = bv[pl.ds(mid, 16)][0]
        go = v < val
        lo2 = jnp.where(go, mid + 1, lo)
        hi2 = jnp.where(go, hi, mid)
        return lo2, hi2
      lo, _ = lax.fori_loop(0, 14, bs, (jnp.int32(0), jnp.int32(_N)))
      return lo

    lb = [jnp.int32(0)]
    for b in range(_NBATCH):
      lb.append(lower_bound(jnp.int32(b + 1)))

    for g in range(_WROWS // 16):
      bvec = bv[pl.ds(r0 + g * 16, 16)]
      svec = jnp.zeros((16,), jnp.int32)
      evec = jnp.zeros((16,), jnp.int32)
      for b in range(_NBATCH):
        svec = jnp.where(bvec == b, lb[b], svec)
        evec = jnp.where(bvec == b, lb[b + 1], evec)
      s_arr[pl.ds(g * 16, 16)] = svec
      e_arr[pl.ds(g * 16, 16)] = evec

    def row_body(r, carry):
      @pl.when(r < cnt)
      def _():
        i = r0 + r
        s = s_arr[pl.ds(r, 16)][0]
        e = e_arr[pl.ds(r, 16)][0]
        cxi = cxv[pl.ds(i, 16)][0]
        cyi = cyv[pl.ds(i, 16)][0]
        czi = czv[pl.ds(i, 16)][0]
        t0 = s // 16
        t1 = (e + 15) // 16
        seg_len = e - s

        def scan_blk(t, cvec):
          j0 = t * 16
          jv = j0 + iota
          x = cxv[pl.ds(j0, 16)]
          y = cyv[pl.ds(j0, 16)]
          z = czv[pl.ds(j0, 16)]
          dx = x - cxi
          dy = y - cyi
          dz = z - czi
          d2 = dx * dx + dy * dy + dz * dz
          inseg = (jv - s).astype(jnp.uint32) < seg_len.astype(jnp.uint32)
          m = inseg & (jv != i) & (d2 < _R2)
          m32 = m.astype(jnp.int32)
          cums = plsc.cumsum(m32)
          pos = cvec + cums - m32
          plsc.store_scatter(bufd, [pos], d2, mask=m)
          plsc.store_scatter(bufi, [pos], jv, mask=m)
          return cvec + plsc.all_reduce_population_count(m)

        C = plsc.parallel_loop(
            t0, t1, 1, unroll=2,
            carry=jnp.zeros((16,), jnp.int32))(scan_blk)[0]
        kc = jnp.minimum(C, _MAXNBR)

        inf_v = jnp.full((16,), _INF, jnp.float32)
        big_v = jnp.full((16,), _IBIG, jnp.int32)
        for p in range(3):
          bufd[pl.ds(C + p * 16, 16)] = inf_v
          bufi[pl.ds(C + p * 16, 16)] = big_v

        k0, v0 = plsc.sort_key_val(bufd[pl.ds(0, 16)], bufi[pl.ds(0, 16)])
        k1, v1 = plsc.sort_key_val(bufd[pl.ds(16, 16)], bufi[pl.ds(16, 16)])
        lo_k, lo_v, hi_k, hi_v = _bitonic_split(k0, v0, k1, v1)
        k0, v0 = plsc.sort_key_val(lo_k, lo_v)
        k1, v1 = plsc.sort_key_val(hi_k, hi_v)

        nblk = (C + 15) // 16

        def sel_blk(t, sel):
          s0, w0, s1, w1 = sel
          sk, sv = plsc.sort_key_val(bufd[pl.ds(t * 16, 16)],
                                     bufi[pl.ds(t * 16, 16)])
          a_k, a_v, rest_k, rest_v = _bitonic_split(s0, w0, sk, sv)
          s0n, w0n = plsc.sort_key_val(a_k, a_v)
          rk, rv = plsc.sort_key_val(rest_k, rest_v)
          c_k, c_v, _, _ = _bitonic_split(s1, w1, rk, rv)
          s1n, w1n = plsc.sort_key_val(c_k, c_v)
          return s0n, w0n, s1n, w1n

        k0, v0, k1, v1 = lax.fori_loop(2, nblk, sel_blk, (k0, v0, k1, v1))

        srcb[pl.ds(r * 32, 16)] = jnp.where(iota < kc, v0, -1)
        srcb[pl.ds(r * 32 + 16, 16)] = jnp.where(iota + 16 < kc, v1, -1)
        dstb[pl.ds(r * 32, 16)] = jnp.where(iota < kc, i, -1)
        dstb[pl.ds(r * 32 + 16, 16)] = jnp.where(iota + 16 < kc, i, -1)
        plsc.store_scatter(degb, [jnp.broadcast_to(r, (16,))],
                           jnp.broadcast_to(kc, (16,)), mask=iota == 0)
      return carry

    lax.fori_loop(0, _WROWS, row_body, 0)

    pltpu.sync_copy(srcb, src_h.at[pl.ds(r0 * 32, _WROWS * 32)])
    pltpu.sync_copy(dstb, dst_h.at[pl.ds(r0 * 32, _WROWS * 32)])
    pltpu.sync_copy(degb, deg_h.at[pl.ds(r0, _WROWS)])

  return _radius_sc


def kernel(node_coord_src, node_feature_src, batch_src):
  cx = node_coord_src[:, 0]
  cy = node_coord_src[:, 1]
  cz = node_coord_src[:, 2]
  src_p, dst_p, deg_p = _build_radius_sc()(cx, cy, cz, batch_src)
  edge_src = src_p[: _N * _MAXNBR]
  edge_dst = dst_p[: _N * _MAXNBR]
  degree = deg_p[:_N]
  return (node_feature_src, node_coord_src, edge_src, edge_dst, degree,
          batch_src)


# speedup vs baseline: 1.3593x; 1.0157x over previous
import functools

import jax
import jax.numpy as jnp
from jax import lax
from jax.experimental import pallas as pl
from jax.experimental.pallas import tpu as pltpu
from jax.experimental.pallas import tpu_sc as plsc

_R2 = 0.15 * 0.15
_MAXNBR = 32
_N = 10000
_NBATCH = 8
_NC = 2
_NS = 16
_NW = _NC * _NS
_WROWS = 320
_NPAD = _NW * _WROWS
_BUF = _N + 64

_INF = float("inf")
_IBIG = 2**31 - 1


def _lex_less(ka, va, kb, vb):
  return (ka < kb) | ((ka == kb) & (va < vb))


def _bitonic_split(ak, av, bk, bv):
  rbk = lax.rev(bk, (0,))
  rbv = lax.rev(bv, (0,))
  m = _lex_less(ak, av, rbk, rbv)
  lo_k = jnp.where(m, ak, rbk)
  lo_v = jnp.where(m, av, rbv)
  hi_k = jnp.where(m, rbk, ak)
  hi_v = jnp.where(m, rbv, av)
  return lo_k, lo_v, hi_k, hi_v


@functools.cache
def _build_radius_sc():
  mesh = plsc.VectorSubcoreMesh(core_axis_name="c", subcore_axis_name="s")

  @functools.partial(
      pl.kernel,
      out_type=(
          jax.ShapeDtypeStruct((_NPAD * _MAXNBR,), jnp.int32),
          jax.ShapeDtypeStruct((_NPAD * _MAXNBR,), jnp.int32),
          jax.ShapeDtypeStruct((_NPAD,), jnp.int32),
      ),
      mesh=mesh,
      scratch_types=[
          pltpu.VMEM((_N + 64,), jnp.float32),
          pltpu.VMEM((_N + 64,), jnp.float32),
          pltpu.VMEM((_N + 64,), jnp.float32),
          pltpu.VMEM((_NPAD,), jnp.int32),
          pltpu.VMEM((_WROWS + 16,), jnp.int32),
          pltpu.VMEM((_WROWS + 16,), jnp.int32),
          pltpu.VMEM((_BUF,), jnp.float32),
          pltpu.VMEM((_BUF,), jnp.int32),
          pltpu.VMEM((_WROWS * _MAXNBR,), jnp.int32),
          pltpu.VMEM((_WROWS * _MAXNBR,), jnp.int32),
          pltpu.VMEM((_WROWS,), jnp.int32),
      ],
      compiler_params=pltpu.CompilerParams(needs_layout_passes=False),
  )
  def _radius_sc(cx_h, cy_h, cz_h, b_h, src_h, dst_h, deg_h,
                 cxv, cyv, czv, bv, s_arr, e_arr, bufd, bufi,
                 srcb, dstb, degb):
    wid = lax.axis_index("s") * _NC + lax.axis_index("c")
    r0 = wid * _WROWS
    cnt = jnp.minimum(_WROWS, _N - r0)

    pltpu.sync_copy(cx_h, cxv.at[pl.ds(0, _N)])
    pltpu.sync_copy(cy_h, cyv.at[pl.ds(0, _N)])
    pltpu.sync_copy(cz_h, czv.at[pl.ds(0, _N)])
    pltpu.sync_copy(b_h, bv.at[pl.ds(0, _N)])

    iota = lax.iota(jnp.int32, 16)

    zf = jnp.zeros((16,), jnp.float32)
    bigb = jnp.full((16,), _NBATCH, jnp.int32)
    for q in range(_N, _N + 64, 16):
      cxv[pl.ds(q, 16)] = zf
      cyv[pl.ds(q, 16)] = zf
      czv[pl.ds(q, 16)] = zf
    for q in range(_N, _NPAD, 16):
      bv[pl.ds(q, 16)] = bigb

    def lower_bound(val):
      def bs(_, carry):
        lo, hi = carry
        mid = (lo + hi) // 2
        v = bv[pl.ds(mid, 16)][0]
        go = v < val
        lo2 = jnp.where(go, mid + 1, lo)
        hi2 = jnp.where(go, hi, mid)
        return lo2, hi2
      lo, _ = lax.fori_loop(0, 14, bs, (jnp.int32(0), jnp.int32(_N)))
      return lo

    lb = [jnp.int32(0)]
    for b in range(_NBATCH):
      lb.append(lower_bound(jnp.int32(b + 1)))

    for g in range(_WROWS // 16):
      bvec = bv[pl.ds(r0 + g * 16, 16)]
      svec = jnp.zeros((16,), jnp.int32)
      evec = jnp.zeros((16,), jnp.int32)
      for b in range(_NBATCH):
        svec = jnp.where(bvec == b, lb[b], svec)
        evec = jnp.where(bvec == b, lb[b + 1], evec)
      s_arr[pl.ds(g * 16, 16)] = svec
      e_arr[pl.ds(g * 16, 16)] = evec

    def row_body(r, carry):
      @pl.when(r < cnt)
      def _():
        i = r0 + r
        s = s_arr[pl.ds(r, 16)][0]
        e = e_arr[pl.ds(r, 16)][0]
        cxi = cxv[pl.ds(i, 16)][0]
        cyi = cyv[pl.ds(i, 16)][0]
        czi = czv[pl.ds(i, 16)][0]
        t0 = s // 16
        t1 = (e + 15) // 16
        seg_len = e - s

        def emit(j0, jv, d2, m, cvec):
          m32 = m.astype(jnp.int32)
          cums = plsc.cumsum(m32)
          pos = cvec + cums - m32
          plsc.store_scatter(bufd, [pos], d2, mask=m)
          plsc.store_scatter(bufi, [pos], jv, mask=m)
          return cvec + plsc.all_reduce_population_count(m)

        def dist2(j0):
          x = cxv[pl.ds(j0, 16)]
          y = cyv[pl.ds(j0, 16)]
          z = czv[pl.ds(j0, 16)]
          dx = x - cxi
          dy = y - cyi
          dz = z - czi
          return dx * dx + dy * dy + dz * dz

        def full_blk(t, cvec, valid):
          j0 = t * 16
          jv = j0 + iota
          d2 = dist2(j0)
          inseg = (jv - s).astype(jnp.uint32) < seg_len.astype(jnp.uint32)
          m = inseg & (jv != i) & (d2 < _R2) & valid
          return emit(j0, jv, d2, m, cvec)

        def fast_blk(t, cvec):
          j0 = t * 16
          d2 = dist2(j0)
          m = d2 < _R2
          return emit(j0, j0 + iota, d2, m, cvec)

        ti = i // 16
        cvec = jnp.zeros((16,), jnp.int32)
        cvec = full_blk(t0, cvec, True)
        cvec = full_blk(t1 - 1, cvec, t1 - 1 != t0)
        cvec = full_blk(ti, cvec, (ti != t0) & (ti != t1 - 1))
        hi_a = jnp.maximum(jnp.minimum(ti, t1 - 1), t0 + 1)
        lo_b = jnp.maximum(ti + 1, t0 + 1)
        hi_b = jnp.maximum(t1 - 1, lo_b)
        cvec = plsc.parallel_loop(
            t0 + 1, hi_a, 1, unroll=4, carry=cvec)(fast_blk)
        cvec = plsc.parallel_loop(
            lo_b, hi_b, 1, unroll=4, carry=cvec)(fast_blk)
        C = cvec[0]
        kc = jnp.minimum(C, _MAXNBR)

        inf_v = jnp.full((16,), _INF, jnp.float32)
        big_v = jnp.full((16,), _IBIG, jnp.int32)
        for p in range(3):
          bufd[pl.ds(C + p * 16, 16)] = inf_v
          bufi[pl.ds(C + p * 16, 16)] = big_v

        k0, v0 = plsc.sort_key_val(bufd[pl.ds(0, 16)], bufi[pl.ds(0, 16)])
        k1, v1 = plsc.sort_key_val(bufd[pl.ds(16, 16)], bufi[pl.ds(16, 16)])
        lo_k, lo_v, hi_k, hi_v = _bitonic_split(k0, v0, k1, v1)
        k0, v0 = plsc.sort_key_val(lo_k, lo_v)
        k1, v1 = plsc.sort_key_val(hi_k, hi_v)

        nblk = (C + 15) // 16

        def sel_blk(t, sel):
          s0, w0, s1, w1 = sel
          sk, sv = plsc.sort_key_val(bufd[pl.ds(t * 16, 16)],
                                     bufi[pl.ds(t * 16, 16)])
          a_k, a_v, rest_k, rest_v = _bitonic_split(s0, w0, sk, sv)
          s0n, w0n = plsc.sort_key_val(a_k, a_v)
          rk, rv = plsc.sort_key_val(rest_k, rest_v)
          c_k, c_v, _, _ = _bitonic_split(s1, w1, rk, rv)
          s1n, w1n = plsc.sort_key_val(c_k, c_v)
          return s0n, w0n, s1n, w1n

        k0, v0, k1, v1 = lax.fori_loop(2, nblk, sel_blk, (k0, v0, k1, v1))

        srcb[pl.ds(r * 32, 16)] = jnp.where(iota < kc, v0, -1)
        srcb[pl.ds(r * 32 + 16, 16)] = jnp.where(iota + 16 < kc, v1, -1)
        dstb[pl.ds(r * 32, 16)] = jnp.where(iota < kc, i, -1)
        dstb[pl.ds(r * 32 + 16, 16)] = jnp.where(iota + 16 < kc, i, -1)
        plsc.store_scatter(degb, [jnp.broadcast_to(r, (16,))],
                           jnp.broadcast_to(kc, (16,)), mask=iota == 0)
      return carry

    lax.fori_loop(0, _WROWS, row_body, 0)

    pltpu.sync_copy(srcb, src_h.at[pl.ds(r0 * 32, _WROWS * 32)])
    pltpu.sync_copy(dstb, dst_h.at[pl.ds(r0 * 32, _WROWS * 32)])
    pltpu.sync_copy(degb, deg_h.at[pl.ds(r0, _WROWS)])

  return _radius_sc


def kernel(node_coord_src, node_feature_src, batch_src):
  cx = node_coord_src[:, 0]
  cy = node_coord_src[:, 1]
  cz = node_coord_src[:, 2]
  src_p, dst_p, deg_p = _build_radius_sc()(cx, cy, cz, batch_src)
  edge_src = src_p[: _N * _MAXNBR]
  edge_dst = dst_p[: _N * _MAXNBR]
  degree = deg_p[:_N]
  return (node_feature_src, node_coord_src, edge_src, edge_dst, degree,
          batch_src)


# speedup vs baseline: 1.3902x; 1.0227x over previous
import functools

import jax
import jax.numpy as jnp
from jax import lax
from jax.experimental import pallas as pl
from jax.experimental.pallas import tpu as pltpu
from jax.experimental.pallas import tpu_sc as plsc

_R2 = 0.15 * 0.15
_MAXNBR = 32
_N = 10000
_NBATCH = 8
_NC = 2
_NS = 16
_NW = _NC * _NS
_WROWS = 320
_NPAD = _NW * _WROWS
_BUF = _N + 64

_INF = float("inf")
_IBIG = 2**31 - 1


def _lex_less(ka, va, kb, vb):
  return (ka < kb) | ((ka == kb) & (va < vb))


def _bitonic_split(ak, av, bk, bv):
  rbk = lax.rev(bk, (0,))
  rbv = lax.rev(bv, (0,))
  m = _lex_less(ak, av, rbk, rbv)
  lo_k = jnp.where(m, ak, rbk)
  lo_v = jnp.where(m, av, rbv)
  hi_k = jnp.where(m, rbk, ak)
  hi_v = jnp.where(m, rbv, av)
  return lo_k, lo_v, hi_k, hi_v


@functools.cache
def _build_radius_sc():
  mesh = plsc.VectorSubcoreMesh(core_axis_name="c", subcore_axis_name="s")

  @functools.partial(
      pl.kernel,
      out_type=(
          jax.ShapeDtypeStruct((_NPAD * _MAXNBR,), jnp.int32),
          jax.ShapeDtypeStruct((_NPAD * _MAXNBR,), jnp.int32),
          jax.ShapeDtypeStruct((_NPAD,), jnp.int32),
      ),
      mesh=mesh,
      scratch_types=[
          pltpu.VMEM((_N + 64,), jnp.float32),
          pltpu.VMEM((_N + 64,), jnp.float32),
          pltpu.VMEM((_N + 64,), jnp.float32),
          pltpu.VMEM((_NPAD,), jnp.int32),
          pltpu.VMEM((_WROWS + 16,), jnp.int32),
          pltpu.VMEM((_WROWS + 16,), jnp.int32),
          pltpu.VMEM((_BUF,), jnp.float32),
          pltpu.VMEM((_BUF,), jnp.int32),
          pltpu.VMEM((_WROWS * _MAXNBR,), jnp.int32),
          pltpu.VMEM((_WROWS * _MAXNBR,), jnp.int32),
          pltpu.VMEM((_WROWS,), jnp.int32),
      ],
      compiler_params=pltpu.CompilerParams(needs_layout_passes=False),
  )
  def _radius_sc(cx_h, cy_h, cz_h, b_h, src_h, dst_h, deg_h,
                 cxv, cyv, czv, bv, s_arr, e_arr, bufd, bufi,
                 srcb, dstb, degb):
    wid = lax.axis_index("s") * _NC + lax.axis_index("c")
    r0 = wid * _WROWS
    cnt = jnp.minimum(_WROWS, _N - r0)

    pltpu.sync_copy(cx_h, cxv.at[pl.ds(0, _N)])
    pltpu.sync_copy(cy_h, cyv.at[pl.ds(0, _N)])
    pltpu.sync_copy(cz_h, czv.at[pl.ds(0, _N)])
    pltpu.sync_copy(b_h, bv.at[pl.ds(0, _N)])

    iota = lax.iota(jnp.int32, 16)

    zf = jnp.zeros((16,), jnp.float32)
    bigb = jnp.full((16,), _NBATCH, jnp.int32)
    for q in range(_N, _N + 64, 16):
      cxv[pl.ds(q, 16)] = zf
      cyv[pl.ds(q, 16)] = zf
      czv[pl.ds(q, 16)] = zf
    for q in range(_N, _NPAD, 16):
      bv[pl.ds(q, 16)] = bigb

    def lower_bound(val):
      def bs(_, carry):
        lo, hi = carry
        mid = (lo + hi) // 2
        v = bv[pl.ds(mid, 16)][0]
        go = v < val
        lo2 = jnp.where(go, mid + 1, lo)
        hi2 = jnp.where(go, hi, mid)
        return lo2, hi2
      lo, _ = lax.fori_loop(0, 14, bs, (jnp.int32(0), jnp.int32(_N)))
      return lo

    lb = [jnp.int32(0)]
    for b in range(_NBATCH):
      lb.append(lower_bound(jnp.int32(b + 1)))

    for g in range(_WROWS // 16):
      bvec = bv[pl.ds(r0 + g * 16, 16)]
      svec = jnp.zeros((16,), jnp.int32)
      evec = jnp.zeros((16,), jnp.int32)
      for b in range(_NBATCH):
        svec = jnp.where(bvec == b, lb[b], svec)
        evec = jnp.where(bvec == b, lb[b + 1], evec)
      s_arr[pl.ds(g * 16, 16)] = svec
      e_arr[pl.ds(g * 16, 16)] = evec

    def row_body(r, carry):
      @pl.when(r < cnt)
      def _():
        i = r0 + r
        s = s_arr[pl.ds(r, 16)][0]
        e = e_arr[pl.ds(r, 16)][0]
        cxi = cxv[pl.ds(i, 16)][0]
        cyi = cyv[pl.ds(i, 16)][0]
        czi = czv[pl.ds(i, 16)][0]
        t0 = s // 16
        t1 = (e + 15) // 16
        seg_len = e - s

        def emit(j0, jv, d2, m, cvec):
          m32 = m.astype(jnp.int32)
          cums = plsc.cumsum(m32)
          pos = cvec + cums - m32
          plsc.store_scatter(bufd, [pos], d2, mask=m)
          plsc.store_scatter(bufi, [pos], jv, mask=m)
          return cvec + plsc.all_reduce_population_count(m)

        def dist2(j0):
          x = cxv[pl.ds(j0, 16)]
          y = cyv[pl.ds(j0, 16)]
          z = czv[pl.ds(j0, 16)]
          dx = x - cxi
          dy = y - cyi
          dz = z - czi
          return dx * dx + dy * dy + dz * dz

        def full_blk(t, cvec, valid):
          j0 = t * 16
          jv = j0 + iota
          d2 = dist2(j0)
          inseg = (jv - s).astype(jnp.uint32) < seg_len.astype(jnp.uint32)
          m = inseg & (jv != i) & (d2 < _R2) & valid
          return emit(j0, jv, d2, m, cvec)

        def fast_blk(t, cvec):
          j0 = t * 16
          jv = j0 + iota
          d2 = dist2(j0)
          m = (jv != i) & (d2 < _R2)
          return emit(j0, jv, d2, m, cvec)

        cvec = jnp.zeros((16,), jnp.int32)
        cvec = full_blk(t0, cvec, True)
        cvec = full_blk(t1 - 1, cvec, t1 - 1 != t0)
        cvec = plsc.parallel_loop(
            t0 + 1, jnp.maximum(t1 - 1, t0 + 1), 1, unroll=4,
            carry=cvec)(fast_blk)
        C = cvec[0]
        kc = jnp.minimum(C, _MAXNBR)

        inf_v = jnp.full((16,), _INF, jnp.float32)
        big_v = jnp.full((16,), _IBIG, jnp.int32)
        for p in range(3):
          bufd[pl.ds(C + p * 16, 16)] = inf_v
          bufi[pl.ds(C + p * 16, 16)] = big_v

        k0, v0 = plsc.sort_key_val(bufd[pl.ds(0, 16)], bufi[pl.ds(0, 16)])
        k1, v1 = plsc.sort_key_val(bufd[pl.ds(16, 16)], bufi[pl.ds(16, 16)])
        lo_k, lo_v, hi_k, hi_v = _bitonic_split(k0, v0, k1, v1)
        k0, v0 = plsc.sort_key_val(lo_k, lo_v)
        k1, v1 = plsc.sort_key_val(hi_k, hi_v)

        nblk = (C + 15) // 16

        def sel_blk(t, sel):
          s0, w0, s1, w1 = sel
          sk, sv = plsc.sort_key_val(bufd[pl.ds(t * 16, 16)],
                                     bufi[pl.ds(t * 16, 16)])
          a_k, a_v, rest_k, rest_v = _bitonic_split(s0, w0, sk, sv)
          s0n, w0n = plsc.sort_key_val(a_k, a_v)
          rk, rv = plsc.sort_key_val(rest_k, rest_v)
          c_k, c_v, _, _ = _bitonic_split(s1, w1, rk, rv)
          s1n, w1n = plsc.sort_key_val(c_k, c_v)
          return s0n, w0n, s1n, w1n

        k0, v0, k1, v1 = lax.fori_loop(2, nblk, sel_blk, (k0, v0, k1, v1))

        srcb[pl.ds(r * 32, 16)] = jnp.where(iota < kc, v0, -1)
        srcb[pl.ds(r * 32 + 16, 16)] = jnp.where(iota + 16 < kc, v1, -1)
        dstb[pl.ds(r * 32, 16)] = jnp.where(iota < kc, i, -1)
        dstb[pl.ds(r * 32 + 16, 16)] = jnp.where(iota + 16 < kc, i, -1)
        plsc.store_scatter(degb, [jnp.broadcast_to(r, (16,))],
                           jnp.broadcast_to(kc, (16,)), mask=iota == 0)
      return carry

    lax.fori_loop(0, _WROWS, row_body, 0)

    pltpu.sync_copy(srcb, src_h.at[pl.ds(r0 * 32, _WROWS * 32)])
    pltpu.sync_copy(dstb, dst_h.at[pl.ds(r0 * 32, _WROWS * 32)])
    pltpu.sync_copy(degb, deg_h.at[pl.ds(r0, _WROWS)])

  return _radius_sc


def kernel(node_coord_src, node_feature_src, batch_src):
  cx = node_coord_src[:, 0]
  cy = node_coord_src[:, 1]
  cz = node_coord_src[:, 2]
  src_p, dst_p, deg_p = _build_radius_sc()(cx, cy, cz, batch_src)
  edge_src = src_p[: _N * _MAXNBR]
  edge_dst = dst_p[: _N * _MAXNBR]
  degree = deg_p[:_N]
  return (node_feature_src, node_coord_src, edge_src, edge_dst, degree,
          batch_src)


# speedup vs baseline: 1.4211x; 1.0222x over previous
import functools

import jax
import jax.numpy as jnp
from jax import lax
from jax.experimental import pallas as pl
from jax.experimental.pallas import tpu as pltpu
from jax.experimental.pallas import tpu_sc as plsc

_R2 = 0.15 * 0.15
_MAXNBR = 32
_N = 10000
_NBATCH = 8
_NC = 2
_NS = 16
_NW = _NC * _NS
_WROWS = 320
_NPAD = _NW * _WROWS
_BUF = _N + 64

_INF = float("inf")
_IBIG = 2**31 - 1


def _lex_less(ka, va, kb, vb):
  return (ka < kb) | ((ka == kb) & (va < vb))


def _bitonic_split(ak, av, bk, bv):
  rbk = lax.rev(bk, (0,))
  rbv = lax.rev(bv, (0,))
  m = _lex_less(ak, av, rbk, rbv)
  lo_k = jnp.where(m, ak, rbk)
  lo_v = jnp.where(m, av, rbv)
  hi_k = jnp.where(m, rbk, ak)
  hi_v = jnp.where(m, rbv, av)
  return lo_k, lo_v, hi_k, hi_v


@functools.cache
def _build_radius_sc():
  mesh = plsc.VectorSubcoreMesh(core_axis_name="c", subcore_axis_name="s")

  @functools.partial(
      pl.kernel,
      out_type=(
          jax.ShapeDtypeStruct((_NPAD * _MAXNBR,), jnp.int32),
          jax.ShapeDtypeStruct((_NPAD * _MAXNBR,), jnp.int32),
          jax.ShapeDtypeStruct((_NPAD,), jnp.int32),
      ),
      mesh=mesh,
      scratch_types=[
          pltpu.VMEM((_N + 64,), jnp.float32),
          pltpu.VMEM((_N + 64,), jnp.float32),
          pltpu.VMEM((_N + 64,), jnp.float32),
          pltpu.VMEM((_NPAD,), jnp.int32),
          pltpu.VMEM((_WROWS + 16,), jnp.int32),
          pltpu.VMEM((_WROWS + 16,), jnp.int32),
          pltpu.VMEM((_BUF,), jnp.float32),
          pltpu.VMEM((_BUF,), jnp.int32),
          pltpu.VMEM((_WROWS * _MAXNBR,), jnp.int32),
          pltpu.VMEM((_WROWS * _MAXNBR,), jnp.int32),
          pltpu.VMEM((_WROWS,), jnp.int32),
      ],
      compiler_params=pltpu.CompilerParams(needs_layout_passes=False),
  )
  def _radius_sc(cx_h, cy_h, cz_h, b_h, src_h, dst_h, deg_h,
                 cxv, cyv, czv, bv, s_arr, e_arr, bufd, bufi,
                 srcb, dstb, degb):
    wid = lax.axis_index("s") * _NC + lax.axis_index("c")
    r0 = wid * _WROWS
    cnt = jnp.minimum(_WROWS, _N - r0)

    pltpu.sync_copy(cx_h, cxv.at[pl.ds(0, _N)])
    pltpu.sync_copy(cy_h, cyv.at[pl.ds(0, _N)])
    pltpu.sync_copy(cz_h, czv.at[pl.ds(0, _N)])
    pltpu.sync_copy(b_h, bv.at[pl.ds(0, _N)])

    iota = lax.iota(jnp.int32, 16)

    zf = jnp.zeros((16,), jnp.float32)
    bigb = jnp.full((16,), _NBATCH, jnp.int32)
    for q in range(_N, _N + 64, 16):
      cxv[pl.ds(q, 16)] = zf
      cyv[pl.ds(q, 16)] = zf
      czv[pl.ds(q, 16)] = zf
    for q in range(_N, _NPAD, 16):
      bv[pl.ds(q, 16)] = bigb

    def lower_bound(val):
      def bs(_, carry):
        lo, hi = carry
        mid = (lo + hi) // 2
        v = bv[pl.ds(mid, 16)][0]
        go = v < val
        lo2 = jnp.where(go, mid + 1, lo)
        hi2 = jnp.where(go, hi, mid)
        return lo2, hi2
      lo, _ = lax.fori_loop(0, 14, bs, (jnp.int32(0), jnp.int32(_N)))
      return lo

    lb = [jnp.int32(0)]
    for b in range(_NBATCH):
      lb.append(lower_bound(jnp.int32(b + 1)))

    for g in range(_WROWS // 16):
      bvec = bv[pl.ds(r0 + g * 16, 16)]
      svec = jnp.zeros((16,), jnp.int32)
      evec = jnp.zeros((16,), jnp.int32)
      for b in range(_NBATCH):
        svec = jnp.where(bvec == b, lb[b], svec)
        evec = jnp.where(bvec == b, lb[b + 1], evec)
      s_arr[pl.ds(g * 16, 16)] = svec
      e_arr[pl.ds(g * 16, 16)] = evec

    def row_body(r, carry):
      @pl.when(r < cnt)
      def _():
        i = r0 + r
        s = s_arr[pl.ds(r, 16)][0]
        e = e_arr[pl.ds(r, 16)][0]
        cxi = cxv[pl.ds(i, 16)][0]
        cyi = cyv[pl.ds(i, 16)][0]
        czi = czv[pl.ds(i, 16)][0]
        t0 = s // 16
        t1 = (e + 15) // 16
        seg_len = e - s

        def emit(j0, jv, d2, m, cvec):
          m32 = m.astype(jnp.int32)
          cums = plsc.cumsum(m32)
          pos = cvec + cums - m32
          plsc.store_scatter(bufd, [pos], d2, mask=m)
          plsc.store_scatter(bufi, [pos], jv, mask=m)
          return cvec + plsc.all_reduce_population_count(m)

        def dist2(j0):
          x = cxv[pl.ds(j0, 16)]
          y = cyv[pl.ds(j0, 16)]
          z = czv[pl.ds(j0, 16)]
          dx = x - cxi
          dy = y - cyi
          dz = z - czi
          return dx * dx + dy * dy + dz * dz

        def full_blk(t, cvec, valid):
          j0 = t * 16
          jv = j0 + iota
          d2 = dist2(j0)
          inseg = (jv - s).astype(jnp.uint32) < seg_len.astype(jnp.uint32)
          m = inseg & (jv != i) & (d2 < _R2) & valid
          return emit(j0, jv, d2, m, cvec)

        def scan_blk(t, cvec):
          j0 = t * 16
          jv = j0 + iota
          d2 = dist2(j0)
          inseg = (jv - s).astype(jnp.uint32) < seg_len.astype(jnp.uint32)
          m = inseg & (jv != i) & (d2 < _R2)
          return emit(j0, jv, d2, m, cvec)

        C = plsc.parallel_loop(
            t0, t1, 1, unroll=4,
            carry=jnp.zeros((16,), jnp.int32))(scan_blk)[0]
        kc = jnp.minimum(C, _MAXNBR)

        inf_v = jnp.full((16,), _INF, jnp.float32)
        big_v = jnp.full((16,), _IBIG, jnp.int32)
        for p in range(3):
          bufd[pl.ds(C + p * 16, 16)] = inf_v
          bufi[pl.ds(C + p * 16, 16)] = big_v

        k0, v0 = plsc.sort_key_val(bufd[pl.ds(0, 16)], bufi[pl.ds(0, 16)])
        k1, v1 = plsc.sort_key_val(bufd[pl.ds(16, 16)], bufi[pl.ds(16, 16)])
        lo_k, lo_v, hi_k, hi_v = _bitonic_split(k0, v0, k1, v1)
        k0, v0 = plsc.sort_key_val(lo_k, lo_v)
        k1, v1 = plsc.sort_key_val(hi_k, hi_v)

        nblk = (C + 15) // 16

        def sel_blk(t, sel):
          s0, w0, s1, w1 = sel
          sk, sv = plsc.sort_key_val(bufd[pl.ds(t * 16, 16)],
                                     bufi[pl.ds(t * 16, 16)])
          a_k, a_v, rest_k, rest_v = _bitonic_split(s0, w0, sk, sv)
          s0n, w0n = plsc.sort_key_val(a_k, a_v)
          rk, rv = plsc.sort_key_val(rest_k, rest_v)
          c_k, c_v, _, _ = _bitonic_split(s1, w1, rk, rv)
          s1n, w1n = plsc.sort_key_val(c_k, c_v)
          return s0n, w0n, s1n, w1n

        k0, v0, k1, v1 = lax.fori_loop(2, nblk, sel_blk, (k0, v0, k1, v1))

        srcb[pl.ds(r * 32, 16)] = jnp.where(iota < kc, v0, -1)
        srcb[pl.ds(r * 32 + 16, 16)] = jnp.where(iota + 16 < kc, v1, -1)
        dstb[pl.ds(r * 32, 16)] = jnp.where(iota < kc, i, -1)
        dstb[pl.ds(r * 32 + 16, 16)] = jnp.where(iota + 16 < kc, i, -1)
        plsc.store_scatter(degb, [jnp.broadcast_to(r, (16,))],
                           jnp.broadcast_to(kc, (16,)), mask=iota == 0)
      return carry

    lax.fori_loop(0, _WROWS, row_body, 0)

    pltpu.sync_copy(srcb, src_h.at[pl.ds(r0 * 32, _WROWS * 32)])
    pltpu.sync_copy(dstb, dst_h.at[pl.ds(r0 * 32, _WROWS * 32)])
    pltpu.sync_copy(degb, deg_h.at[pl.ds(r0, _WROWS)])

  return _radius_sc


def kernel(node_coord_src, node_feature_src, batch_src):
  cx = node_coord_src[:, 0]
  cy = node_coord_src[:, 1]
  cz = node_coord_src[:, 2]
  src_p, dst_p, deg_p = _build_radius_sc()(cx, cy, cz, batch_src)
  edge_src = src_p[: _N * _MAXNBR]
  edge_dst = dst_p[: _N * _MAXNBR]
  degree = deg_p[:_N]
  return (node_feature_src, node_coord_src, edge_src, edge_dst, degree,
          batch_src)


# speedup vs baseline: 1.4414x; 1.0143x over previous
import functools

import jax
import jax.numpy as jnp
from jax import lax
from jax.experimental import pallas as pl
from jax.experimental.pallas import tpu as pltpu
from jax.experimental.pallas import tpu_sc as plsc

_R2 = 0.15 * 0.15
_MAXNBR = 32
_N = 10000
_NBATCH = 8
_NC = 2
_NS = 16
_NW = _NC * _NS
_WROWS = 320
_NPAD = _NW * _WROWS
_QBUF = 2512
_BUF = 4 * _QBUF + _N + 64
_OVTH = _QBUF - 16

_INF = float("inf")
_IBIG = 2**31 - 1


def _lex_less(ka, va, kb, vb):
  return (ka < kb) | ((ka == kb) & (va < vb))


def _bitonic_split(ak, av, bk, bv):
  rbk = lax.rev(bk, (0,))
  rbv = lax.rev(bv, (0,))
  m = _lex_less(ak, av, rbk, rbv)
  lo_k = jnp.where(m, ak, rbk)
  lo_v = jnp.where(m, av, rbv)
  hi_k = jnp.where(m, rbk, ak)
  hi_v = jnp.where(m, rbv, av)
  return lo_k, lo_v, hi_k, hi_v


@functools.cache
def _build_radius_sc():
  mesh = plsc.VectorSubcoreMesh(core_axis_name="c", subcore_axis_name="s")

  @functools.partial(
      pl.kernel,
      out_type=(
          jax.ShapeDtypeStruct((_NPAD * _MAXNBR,), jnp.int32),
          jax.ShapeDtypeStruct((_NPAD * _MAXNBR,), jnp.int32),
          jax.ShapeDtypeStruct((_NPAD,), jnp.int32),
      ),
      mesh=mesh,
      scratch_types=[
          pltpu.VMEM((_N + 64,), jnp.float32),
          pltpu.VMEM((_N + 64,), jnp.float32),
          pltpu.VMEM((_N + 64,), jnp.float32),
          pltpu.VMEM((_NPAD,), jnp.int32),
          pltpu.VMEM((_WROWS + 16,), jnp.int32),
          pltpu.VMEM((_WROWS + 16,), jnp.int32),
          pltpu.VMEM((_BUF,), jnp.float32),
          pltpu.VMEM((_BUF,), jnp.int32),
          pltpu.VMEM((_WROWS * _MAXNBR,), jnp.int32),
          pltpu.VMEM((_WROWS * _MAXNBR,), jnp.int32),
          pltpu.VMEM((_WROWS,), jnp.int32),
      ],
      compiler_params=pltpu.CompilerParams(needs_layout_passes=False),
  )
  def _radius_sc(cx_h, cy_h, cz_h, b_h, src_h, dst_h, deg_h,
                 cxv, cyv, czv, bv, s_arr, e_arr, bufd, bufi,
                 srcb, dstb, degb):
    wid = lax.axis_index("s") * _NC + lax.axis_index("c")
    r0 = wid * _WROWS
    cnt = jnp.minimum(_WROWS, _N - r0)

    pltpu.sync_copy(cx_h, cxv.at[pl.ds(0, _N)])
    pltpu.sync_copy(cy_h, cyv.at[pl.ds(0, _N)])
    pltpu.sync_copy(cz_h, czv.at[pl.ds(0, _N)])
    pltpu.sync_copy(b_h, bv.at[pl.ds(0, _N)])

    iota = lax.iota(jnp.int32, 16)

    zf = jnp.zeros((16,), jnp.float32)
    bigb = jnp.full((16,), _NBATCH, jnp.int32)
    for q in range(_N, _N + 64, 16):
      cxv[pl.ds(q, 16)] = zf
      cyv[pl.ds(q, 16)] = zf
      czv[pl.ds(q, 16)] = zf
    for q in range(_N, _NPAD, 16):
      bv[pl.ds(q, 16)] = bigb

    def lower_bound(val):
      def bs(_, carry):
        lo, hi = carry
        mid = (lo + hi) // 2
        v = bv[pl.ds(mid, 16)][0]
        go = v < val
        lo2 = jnp.where(go, mid + 1, lo)
        hi2 = jnp.where(go, hi, mid)
        return lo2, hi2
      lo, _ = lax.fori_loop(0, 14, bs, (jnp.int32(0), jnp.int32(_N)))
      return lo

    lb = [jnp.int32(0)]
    for b in range(_NBATCH):
      lb.append(lower_bound(jnp.int32(b + 1)))

    for g in range(_WROWS // 16):
      bvec = bv[pl.ds(r0 + g * 16, 16)]
      svec = jnp.zeros((16,), jnp.int32)
      evec = jnp.zeros((16,), jnp.int32)
      for b in range(_NBATCH):
        svec = jnp.where(bvec == b, lb[b], svec)
        evec = jnp.where(bvec == b, lb[b + 1], evec)
      s_arr[pl.ds(g * 16, 16)] = svec
      e_arr[pl.ds(g * 16, 16)] = evec

    def select_write(r, i, cc, qb):
      kc = jnp.minimum(cc, _MAXNBR)
      inf_v = jnp.full((16,), _INF, jnp.float32)
      big_v = jnp.full((16,), _IBIG, jnp.int32)
      for p in range(3):
        bufd[pl.ds(qb + cc + p * 16, 16)] = inf_v
        bufi[pl.ds(qb + cc + p * 16, 16)] = big_v

      k0, v0 = plsc.sort_key_val(bufd[pl.ds(qb, 16)], bufi[pl.ds(qb, 16)])
      k1, v1 = plsc.sort_key_val(bufd[pl.ds(qb + 16, 16)],
                                 bufi[pl.ds(qb + 16, 16)])
      lo_k, lo_v, hi_k, hi_v = _bitonic_split(k0, v0, k1, v1)
      k0, v0 = plsc.sort_key_val(lo_k, lo_v)
      k1, v1 = plsc.sort_key_val(hi_k, hi_v)

      nblk = (cc + 15) // 16

      def sel_blk(t, sel):
        s0, w0, s1, w1 = sel
        sk, sv = plsc.sort_key_val(bufd[pl.ds(qb + t * 16, 16)],
                                   bufi[pl.ds(qb + t * 16, 16)])
        a_k, a_v, rest_k, rest_v = _bitonic_split(s0, w0, sk, sv)
        s0n, w0n = plsc.sort_key_val(a_k, a_v)
        rk, rv = plsc.sort_key_val(rest_k, rest_v)
        c_k, c_v, _, _ = _bitonic_split(s1, w1, rk, rv)
        s1n, w1n = plsc.sort_key_val(c_k, c_v)
        return s0n, w0n, s1n, w1n

      k0, v0, k1, v1 = lax.fori_loop(2, nblk, sel_blk, (k0, v0, k1, v1))

      srcb[pl.ds(r * 32, 16)] = jnp.where(iota < kc, v0, -1)
      srcb[pl.ds(r * 32 + 16, 16)] = jnp.where(iota + 16 < kc, v1, -1)
      dstb[pl.ds(r * 32, 16)] = jnp.where(iota < kc, i, -1)
      dstb[pl.ds(r * 32 + 16, 16)] = jnp.where(iota + 16 < kc, i, -1)
      plsc.store_scatter(degb, [jnp.broadcast_to(r, (16,))],
                         jnp.broadcast_to(kc, (16,)), mask=iota == 0)

    def grp_body(g, carry):
      @pl.when(g * 4 < cnt)
      def _():
        base_r = g * 4
        i0 = r0 + base_r
        sv = s_arr[pl.ds(base_r, 16)]
        ev = e_arr[pl.ds(base_r, 16)]
        xv = cxv[pl.ds(i0, 16)]
        yv = cyv[pl.ds(i0, 16)]
        zv = czv[pl.ds(i0, 16)]
        S = [sv[q] for q in range(4)]
        E = [ev[q] for q in range(4)]
        CX = [xv[q] for q in range(4)]
        CY = [yv[q] for q in range(4)]
        CZ = [zv[q] for q in range(4)]
        L = [(E[q] - S[q]).astype(jnp.uint32) for q in range(4)]
        I = [i0 + q for q in range(4)]

        s_min = jnp.minimum(jnp.minimum(S[0], S[1]),
                            jnp.minimum(S[2], S[3]))
        e_max = jnp.maximum(jnp.maximum(E[0], E[1]),
                            jnp.maximum(E[2], E[3]))
        gt0 = s_min // 16
        gt1 = (e_max + 15) // 16

        def scan4(t, cs):
          j0 = t * 16
          jv = j0 + iota
          x = cxv[pl.ds(j0, 16)]
          y = cyv[pl.ds(j0, 16)]
          z = czv[pl.ds(j0, 16)]
          outs = []
          for q in range(4):
            dx = x - CX[q]
            dy = y - CY[q]
            dz = z - CZ[q]
            d2 = dx * dx + dy * dy + dz * dz
            inseg = (jv - S[q]).astype(jnp.uint32) < L[q]
            m = inseg & (jv != I[q]) & (d2 < _R2)
            m32 = m.astype(jnp.int32)
            cums = plsc.cumsum(m32)
            pos = (q * _QBUF) + cs[q] + cums - m32
            plsc.store_scatter(bufd, [pos], d2, mask=m)
            plsc.store_scatter(bufi, [pos], jv, mask=m)
            outs.append(cs[q] + plsc.all_reduce_population_count(m))
          return tuple(outs)

        zero = jnp.zeros((16,), jnp.int32)
        cs = plsc.parallel_loop(
            gt0, gt1, 1, unroll=2,
            carry=(zero, zero, zero, zero))(scan4)
        C = [cs[q][0] for q in range(4)]

        ovf = ((C[0] > _OVTH) | (C[1] > _OVTH)
               | (C[2] > _OVTH) | (C[3] > _OVTH))

        @pl.when(jnp.logical_not(ovf))
        def _():
          for q in range(4):
            select_write(base_r + q, I[q], C[q], q * _QBUF)

        @pl.when(ovf)
        def _():
          for q in range(4):
            t0 = S[q] // 16
            t1 = (E[q] + 15) // 16

            def rescan(t, cvec, q=q):
              j0 = t * 16
              jv = j0 + iota
              x = cxv[pl.ds(j0, 16)]
              y = cyv[pl.ds(j0, 16)]
              z = czv[pl.ds(j0, 16)]
              dx = x - CX[q]
              dy = y - CY[q]
              dz = z - CZ[q]
              d2 = dx * dx + dy * dy + dz * dz
              inseg = (jv - S[q]).astype(jnp.uint32) < L[q]
              m = inseg & (jv != I[q]) & (d2 < _R2)
              m32 = m.astype(jnp.int32)
              cums = plsc.cumsum(m32)
              pos = cvec + cums - m32
              plsc.store_scatter(bufd, [pos], d2, mask=m)
              plsc.store_scatter(bufi, [pos], jv, mask=m)
              return cvec + plsc.all_reduce_population_count(m)

            cq = plsc.parallel_loop(
                t0, t1, 1, unroll=4, carry=zero)(rescan)[0]
            select_write(base_r + q, I[q], cq, 0)
      return carry

    lax.fori_loop(0, _WROWS // 4, grp_body, 0)

    pltpu.sync_copy(srcb, src_h.at[pl.ds(r0 * 32, _WROWS * 32)])
    pltpu.sync_copy(dstb, dst_h.at[pl.ds(r0 * 32, _WROWS * 32)])
    pltpu.sync_copy(degb, deg_h.at[pl.ds(r0, _WROWS)])

  return _radius_sc


def kernel(node_coord_src, node_feature_src, batch_src):
  cx = node_coord_src[:, 0]
  cy = node_coord_src[:, 1]
  cz = node_coord_src[:, 2]
  src_p, dst_p, deg_p = _build_radius_sc()(cx, cy, cz, batch_src)
  edge_src = src_p[: _N * _MAXNBR]
  edge_dst = dst_p[: _N * _MAXNBR]
  degree = deg_p[:_N]
  return (node_feature_src, node_coord_src, edge_src, edge_dst, degree,
          batch_src)


# speedup vs baseline: 1.4436x; 1.0015x over previous
import functools

import jax
import jax.numpy as jnp
from jax import lax
from jax.experimental import pallas as pl
from jax.experimental.pallas import tpu as pltpu
from jax.experimental.pallas import tpu_sc as plsc

_R2 = 0.15 * 0.15
_MAXNBR = 32
_N = 10000
_NBATCH = 8
_NC = 2
_NS = 16
_NW = _NC * _NS
_WROWS = 320
_NPAD = _NW * _WROWS
_QBUF = 2512
_BUF = 4 * _QBUF + _N + 64
_OVTH = _QBUF - 16

_INF = float("inf")
_IBIG = 2**31 - 1


def _lex_less(ka, va, kb, vb):
  return (ka < kb) | ((ka == kb) & (va < vb))


def _bitonic_split(ak, av, bk, bv):
  rbk = lax.rev(bk, (0,))
  rbv = lax.rev(bv, (0,))
  m = _lex_less(ak, av, rbk, rbv)
  lo_k = jnp.where(m, ak, rbk)
  lo_v = jnp.where(m, av, rbv)
  hi_k = jnp.where(m, rbk, ak)
  hi_v = jnp.where(m, rbv, av)
  return lo_k, lo_v, hi_k, hi_v


@functools.cache
def _build_radius_sc():
  mesh = plsc.VectorSubcoreMesh(core_axis_name="c", subcore_axis_name="s")

  @functools.partial(
      pl.kernel,
      out_type=(
          jax.ShapeDtypeStruct((_NPAD * _MAXNBR,), jnp.int32),
          jax.ShapeDtypeStruct((_NPAD * _MAXNBR,), jnp.int32),
          jax.ShapeDtypeStruct((_NPAD,), jnp.int32),
      ),
      mesh=mesh,
      scratch_types=[
          pltpu.VMEM((_N + 64,), jnp.float32),
          pltpu.VMEM((_N + 64,), jnp.float32),
          pltpu.VMEM((_N + 64,), jnp.float32),
          pltpu.VMEM((_NPAD,), jnp.int32),
          pltpu.VMEM((_WROWS + 16,), jnp.int32),
          pltpu.VMEM((_WROWS + 16,), jnp.int32),
          pltpu.VMEM((_BUF,), jnp.float32),
          pltpu.VMEM((_BUF,), jnp.int32),
          pltpu.VMEM((_WROWS * _MAXNBR,), jnp.int32),
          pltpu.VMEM((_WROWS * _MAXNBR,), jnp.int32),
          pltpu.VMEM((_WROWS,), jnp.int32),
      ],
      compiler_params=pltpu.CompilerParams(needs_layout_passes=False),
  )
  def _radius_sc(cx_h, cy_h, cz_h, b_h, src_h, dst_h, deg_h,
                 cxv, cyv, czv, bv, s_arr, e_arr, bufd, bufi,
                 srcb, dstb, degb):
    wid = lax.axis_index("s") * _NC + lax.axis_index("c")
    r0 = wid * _WROWS
    cnt = jnp.minimum(_WROWS, _N - r0)

    pltpu.sync_copy(cx_h, cxv.at[pl.ds(0, _N)])
    pltpu.sync_copy(cy_h, cyv.at[pl.ds(0, _N)])
    pltpu.sync_copy(cz_h, czv.at[pl.ds(0, _N)])
    pltpu.sync_copy(b_h, bv.at[pl.ds(0, _N)])

    iota = lax.iota(jnp.int32, 16)

    zf = jnp.zeros((16,), jnp.float32)
    bigb = jnp.full((16,), _NBATCH, jnp.int32)
    for q in range(_N, _N + 64, 16):
      cxv[pl.ds(q, 16)] = zf
      cyv[pl.ds(q, 16)] = zf
      czv[pl.ds(q, 16)] = zf
    for q in range(_N, _NPAD, 16):
      bv[pl.ds(q, 16)] = bigb

    def lower_bound(val):
      def bs(_, carry):
        lo, hi = carry
        mid = (lo + hi) // 2
        v = bv[pl.ds(mid, 16)][0]
        go = v < val
        lo2 = jnp.where(go, mid + 1, lo)
        hi2 = jnp.where(go, hi, mid)
        return lo2, hi2
      lo, _ = lax.fori_loop(0, 14, bs, (jnp.int32(0), jnp.int32(_N)))
      return lo

    lb = [jnp.int32(0)]
    for b in range(_NBATCH):
      lb.append(lower_bound(jnp.int32(b + 1)))

    for g in range(_WROWS // 16):
      bvec = bv[pl.ds(r0 + g * 16, 16)]
      svec = jnp.zeros((16,), jnp.int32)
      evec = jnp.zeros((16,), jnp.int32)
      for b in range(_NBATCH):
        svec = jnp.where(bvec == b, lb[b], svec)
        evec = jnp.where(bvec == b, lb[b + 1], evec)
      s_arr[pl.ds(g * 16, 16)] = svec
      e_arr[pl.ds(g * 16, 16)] = evec

    def select_write(r, i, cc, qb):
      kc = jnp.minimum(cc, _MAXNBR)
      inf_v = jnp.full((16,), _INF, jnp.float32)
      big_v = jnp.full((16,), _IBIG, jnp.int32)
      for p in range(3):
        bufd[pl.ds(qb + cc + p * 16, 16)] = inf_v
        bufi[pl.ds(qb + cc + p * 16, 16)] = big_v

      k0, v0 = plsc.sort_key_val(bufd[pl.ds(qb, 16)], bufi[pl.ds(qb, 16)])
      k1, v1 = plsc.sort_key_val(bufd[pl.ds(qb + 16, 16)],
                                 bufi[pl.ds(qb + 16, 16)])
      lo_k, lo_v, hi_k, hi_v = _bitonic_split(k0, v0, k1, v1)
      k0, v0 = plsc.sort_key_val(lo_k, lo_v)
      k1, v1 = plsc.sort_key_val(hi_k, hi_v)

      nblk = (cc + 15) // 16

      def sel_blk(t, sel):
        s0, w0, s1, w1 = sel
        sk, sv = plsc.sort_key_val(bufd[pl.ds(qb + t * 16, 16)],
                                   bufi[pl.ds(qb + t * 16, 16)])
        a_k, a_v, rest_k, rest_v = _bitonic_split(s0, w0, sk, sv)
        s0n, w0n = plsc.sort_key_val(a_k, a_v)
        rk, rv = plsc.sort_key_val(rest_k, rest_v)
        c_k, c_v, _, _ = _bitonic_split(s1, w1, rk, rv)
        s1n, w1n = plsc.sort_key_val(c_k, c_v)
        return s0n, w0n, s1n, w1n

      k0, v0, k1, v1 = lax.fori_loop(2, nblk, sel_blk, (k0, v0, k1, v1))

      srcb[pl.ds(r * 32, 16)] = jnp.where(iota < kc, v0, -1)
      srcb[pl.ds(r * 32 + 16, 16)] = jnp.where(iota + 16 < kc, v1, -1)
      dstb[pl.ds(r * 32, 16)] = jnp.where(iota < kc, i, -1)
      dstb[pl.ds(r * 32 + 16, 16)] = jnp.where(iota + 16 < kc, i, -1)
      plsc.store_scatter(degb, [jnp.broadcast_to(r, (16,))],
                         jnp.broadcast_to(kc, (16,)), mask=iota == 0)

    def grp_body(g, carry):
      @pl.when(g * 4 < cnt)
      def _():
        base_r = g * 4
        i0 = r0 + base_r
        sv = s_arr[pl.ds(base_r, 16)]
        ev = e_arr[pl.ds(base_r, 16)]
        xv = cxv[pl.ds(i0, 16)]
        yv = cyv[pl.ds(i0, 16)]
        zv = czv[pl.ds(i0, 16)]
        S = [sv[q] for q in range(4)]
        E = [ev[q] for q in range(4)]
        CX = [xv[q] for q in range(4)]
        CY = [yv[q] for q in range(4)]
        CZ = [zv[q] for q in range(4)]
        L = [(E[q] - S[q]).astype(jnp.uint32) for q in range(4)]
        I = [i0 + q for q in range(4)]

        s_min = jnp.minimum(jnp.minimum(S[0], S[1]),
                            jnp.minimum(S[2], S[3]))
        e_max = jnp.maximum(jnp.maximum(E[0], E[1]),
                            jnp.maximum(E[2], E[3]))
        gt0 = s_min // 16
        gt1 = (e_max + 15) // 16

        def scan4(t, cs):
          j0 = t * 16
          jv = j0 + iota
          x = cxv[pl.ds(j0, 16)]
          y = cyv[pl.ds(j0, 16)]
          z = czv[pl.ds(j0, 16)]
          outs = []
          for q in range(4):
            dx = x - CX[q]
            dy = y - CY[q]
            dz = z - CZ[q]
            d2 = dx * dx + dy * dy + dz * dz
            inseg = (jv - S[q]).astype(jnp.uint32) < L[q]
            m = inseg & (jv != I[q]) & (d2 < _R2)
            m32 = m.astype(jnp.int32)
            cums = plsc.cumsum(m32)
            pos = (q * _QBUF) + cs[q] + cums - m32
            plsc.store_scatter(bufd, [pos], d2, mask=m)
            plsc.store_scatter(bufi, [pos], jv, mask=m)
            outs.append(cs[q] + plsc.all_reduce_population_count(m))
          return tuple(outs)

        zero = jnp.zeros((16,), jnp.int32)
        cs = plsc.parallel_loop(
            gt0, gt1, 1, unroll=4,
            carry=(zero, zero, zero, zero))(scan4)
        C = [cs[q][0] for q in range(4)]

        ovf = ((C[0] > _OVTH) | (C[1] > _OVTH)
               | (C[2] > _OVTH) | (C[3] > _OVTH))

        @pl.when(jnp.logical_not(ovf))
        def _():
          for q in range(4):
            select_write(base_r + q, I[q], C[q], q * _QBUF)

        @pl.when(ovf)
        def _():
          for q in range(4):
            t0 = S[q] // 16
            t1 = (E[q] + 15) // 16

            def rescan(t, cvec, q=q):
              j0 = t * 16
              jv = j0 + iota
              x = cxv[pl.ds(j0, 16)]
              y = cyv[pl.ds(j0, 16)]
              z = czv[pl.ds(j0, 16)]
              dx = x - CX[q]
              dy = y - CY[q]
              dz = z - CZ[q]
              d2 = dx * dx + dy * dy + dz * dz
              inseg = (jv - S[q]).astype(jnp.uint32) < L[q]
              m = inseg & (jv != I[q]) & (d2 < _R2)
              m32 = m.astype(jnp.int32)
              cums = plsc.cumsum(m32)
              pos = cvec + cums - m32
              plsc.store_scatter(bufd, [pos], d2, mask=m)
              plsc.store_scatter(bufi, [pos], jv, mask=m)
              return cvec + plsc.all_reduce_population_count(m)

            cq = plsc.parallel_loop(
                t0, t1, 1, unroll=4, carry=zero)(rescan)[0]
            select_write(base_r + q, I[q], cq, 0)
      return carry

    lax.fori_loop(0, _WROWS // 4, grp_body, 0)

    pltpu.sync_copy(srcb, src_h.at[pl.ds(r0 * 32, _WROWS * 32)])
    pltpu.sync_copy(dstb, dst_h.at[pl.ds(r0 * 32, _WROWS * 32)])
    pltpu.sync_copy(degb, deg_h.at[pl.ds(r0, _WROWS)])

  return _radius_sc


def kernel(node_coord_src, node_feature_src, batch_src):
  cx = node_coord_src[:, 0]
  cy = node_coord_src[:, 1]
  cz = node_coord_src[:, 2]
  src_p, dst_p, deg_p = _build_radius_sc()(cx, cy, cz, batch_src)
  edge_src = src_p[: _N * _MAXNBR]
  edge_dst = dst_p[: _N * _MAXNBR]
  degree = deg_p[:_N]
  return (node_feature_src, node_coord_src, edge_src, edge_dst, degree,
          batch_src)
